# Initial kernel scaffold; baseline (speedup 1.0000x reference)
import functools

import jax
import jax.numpy as jnp
from jax import lax
from jax.experimental import pallas as pl
from jax.experimental.pallas import tpu as pltpu
from jax.experimental.pallas import tpu_sc as plsc

N_NODES = 10000
N_EDGES = 320000
D_FEAT = 132
D_PAD = 144
WIDTH = 128

NUM_CORES = 2
NUM_SUBCORES = 16
NW = NUM_CORES * NUM_SUBCORES

CHUNK = 128
K_CHUNKS = 80
E_PAD = NW * K_CHUNKS * CHUNK
ROWS_PER_TILE = 640
N_ACC = NUM_SUBCORES * ROWS_PER_TILE
DUMMY_ROW = N_NODES


def _make_agg_kernel():
  mesh = plsc.VectorSubcoreMesh(
      core_axis_name="c", subcore_axis_name="s",
      num_cores=NUM_CORES, num_subcores=NUM_SUBCORES)

  @functools.partial(
      pl.kernel,
      out_type=jax.ShapeDtypeStruct((NUM_CORES * N_ACC, D_PAD), jnp.float32),
      mesh=mesh,
      scratch_types=[
          pltpu.VMEM((K_CHUNKS, CHUNK), jnp.int32),
          pltpu.VMEM((K_CHUNKS, CHUNK), jnp.int32),
          pltpu.VMEM((CHUNK, D_PAD), jnp.float32),
          pltpu.VMEM_SHARED((N_ACC, D_PAD), jnp.float32),
          pltpu.SemaphoreType.DMA,
      ],
  )
  def agg(feats_hbm, src_hbm, dst_hbm, zeros_hbm, out_hbm,
          src_v, dst_v, rows_v, accum, sem):
    c = lax.axis_index("c")
    s = lax.axis_index("s")
    wid = s * NUM_CORES + c

    pltpu.sync_copy(src_hbm.at[wid], src_v)
    pltpu.sync_copy(dst_hbm.at[wid], dst_v)

    pltpu.sync_copy(zeros_hbm, rows_v)
    for k in range(ROWS_PER_TILE // CHUNK):
      pltpu.sync_copy(
          rows_v, accum.at[pl.ds(s * ROWS_PER_TILE + k * CHUNK, CHUNK)])
    plsc.subcore_barrier()

    def body(j, carry):
      pltpu.async_copy(feats_hbm.at[dst_v.at[j]], rows_v, sem).wait()
      pltpu.sync_copy(rows_v, accum.at[src_v.at[j]], add=True)
      return carry

    lax.fori_loop(0, K_CHUNKS, body, 0)
    plsc.subcore_barrier()

    for k in range(ROWS_PER_TILE // CHUNK):
      base = s * ROWS_PER_TILE + k * CHUNK
      pltpu.sync_copy(accum.at[pl.ds(base, CHUNK)], rows_v)
      pltpu.sync_copy(rows_v, out_hbm.at[pl.ds(c * N_ACC + base, CHUNK)])

  return agg


def _mlp_body(p_ref, w0_ref, b0_ref, w1_ref, b1_ref, w2_ref, b2_ref, o_ref):
  s = p_ref[0] + p_ref[1]
  s = s[:N_NODES]
  nrm2 = jnp.sum(s * s, axis=1, keepdims=True)
  h = s * lax.rsqrt(nrm2)
  h = jnp.maximum(jnp.dot(h, w0_ref[...],
                          preferred_element_type=jnp.float32) + b0_ref[...], 0.0)
  h = jnp.maximum(jnp.dot(h, w1_ref[...],
                          preferred_element_type=jnp.float32) + b1_ref[...], 0.0)
  r = jnp.dot(h, w2_ref[...], preferred_element_type=jnp.float32)
  o_ref[...] = (jnp.sum(r) / N_NODES + b2_ref[0, 0]).reshape(1, 1)


def _mlp(partials, W0p, b0, W1, b1, W2, b2):
  return pl.pallas_call(
      _mlp_body,
      out_shape=jax.ShapeDtypeStruct((1, 1), jnp.float32),
  )(partials, W0p, b0, W1, b1, W2, b2)


@jax.jit
def kernel(x, pos, z, edge_index, W0, b0, W1, b1, W2, b2):
  feats = jnp.zeros((N_NODES, D_PAD), jnp.float32)
  feats = feats.at[:, :128].set(x)
  feats = feats.at[:, 128:131].set(pos)
  feats = feats.at[:, 131].set(z)

  src = edge_index[0].astype(jnp.int32)
  dst = edge_index[1].astype(jnp.int32)
  pad = E_PAD - N_EDGES
  src = jnp.concatenate([src, jnp.full((pad,), DUMMY_ROW, jnp.int32)])
  dst = jnp.concatenate([dst, jnp.zeros((pad,), jnp.int32)])
  src = src.reshape(NW, K_CHUNKS, CHUNK)
  dst = dst.reshape(NW, K_CHUNKS, CHUNK)
  zeros_blk = jnp.zeros((CHUNK, D_PAD), jnp.float32)

  out = _make_agg_kernel()(feats, src, dst, zeros_blk)
  partials = out.reshape(NUM_CORES, N_ACC, D_PAD)

  W0p = jnp.zeros((D_PAD, WIDTH), jnp.float32).at[:D_FEAT].set(W0)
  res = _mlp(partials, W0p, b0.reshape(1, WIDTH), W1, b1.reshape(1, WIDTH),
             W2, b2.reshape(1, 1))
  return res.reshape(1)


# speedup vs baseline: 3.3719x; 3.3719x over previous
import functools

import jax
import jax.numpy as jnp
from jax import lax
from jax.experimental import pallas as pl
from jax.experimental.pallas import tpu as pltpu
from jax.experimental.pallas import tpu_sc as plsc

N_NODES = 10000
N_EDGES = 320000
D_FEAT = 132
D_PAD = 144
WIDTH = 128

NUM_CORES = 2
NUM_SUBCORES = 16
NW = NUM_CORES * NUM_SUBCORES

CHUNK = 128
K_CHUNKS = 80
E_PAD = NW * K_CHUNKS * CHUNK
ROWS_PER_TILE = 640
N_ACC = NUM_SUBCORES * ROWS_PER_TILE
DUMMY_ROW = N_NODES


def _make_agg_kernel():
  mesh = plsc.VectorSubcoreMesh(
      core_axis_name="c", subcore_axis_name="s",
      num_cores=NUM_CORES, num_subcores=NUM_SUBCORES)

  @functools.partial(
      pl.kernel,
      out_type=jax.ShapeDtypeStruct((NUM_CORES * N_ACC, D_PAD), jnp.float32),
      mesh=mesh,
      scratch_types=[
          pltpu.VMEM((K_CHUNKS, CHUNK), jnp.int32),
          pltpu.VMEM((K_CHUNKS, CHUNK), jnp.int32),
          pltpu.VMEM((CHUNK, D_PAD), jnp.float32),
          pltpu.VMEM_SHARED((N_ACC, D_PAD), jnp.float32),
          pltpu.SemaphoreType.DMA,
      ],
      compiler_params=pltpu.CompilerParams(use_tc_tiling_on_sc=False),
  )
  def agg(feats_hbm, src_hbm, dst_hbm, zeros_hbm, out_hbm,
          src_v, dst_v, rows_v, accum, sem):
    c = lax.axis_index("c")
    s = lax.axis_index("s")
    wid = s * NUM_CORES + c

    pltpu.sync_copy(src_hbm.at[wid], src_v)
    pltpu.sync_copy(dst_hbm.at[wid], dst_v)

    pltpu.sync_copy(zeros_hbm, rows_v)
    for k in range(ROWS_PER_TILE // CHUNK):
      pltpu.sync_copy(
          rows_v, accum.at[pl.ds(s * ROWS_PER_TILE + k * CHUNK, CHUNK)])
    plsc.subcore_barrier()

    def body(j, carry):
      pltpu.async_copy(feats_hbm.at[dst_v.at[j]], rows_v, sem).wait()
      pltpu.sync_copy(rows_v, accum.at[src_v.at[j]], add=True)
      return carry

    lax.fori_loop(0, K_CHUNKS, body, 0)
    plsc.subcore_barrier()

    for k in range(ROWS_PER_TILE // CHUNK):
      base = s * ROWS_PER_TILE + k * CHUNK
      pltpu.sync_copy(accum.at[pl.ds(base, CHUNK)], rows_v)
      pltpu.sync_copy(rows_v, out_hbm.at[pl.ds(c * N_ACC + base, CHUNK)])

  return agg


def _mlp_body(p_ref, w0_ref, b0_ref, w1_ref, b1_ref, w2_ref, b2_ref, o_ref):
  s = p_ref[0] + p_ref[1]
  s = s[:N_NODES]
  nrm2 = jnp.sum(s * s, axis=1, keepdims=True)
  h = s * lax.rsqrt(nrm2)
  h = jnp.maximum(jnp.dot(h, w0_ref[...],
                          preferred_element_type=jnp.float32) + b0_ref[...], 0.0)
  h = jnp.maximum(jnp.dot(h, w1_ref[...],
                          preferred_element_type=jnp.float32) + b1_ref[...], 0.0)
  r = jnp.dot(h, w2_ref[...], preferred_element_type=jnp.float32)
  o_ref[...] = (jnp.sum(r) / N_NODES + b2_ref[0, 0]).reshape(1, 1)


def _mlp(partials, W0p, b0, W1, b1, W2, b2):
  return pl.pallas_call(
      _mlp_body,
      out_shape=jax.ShapeDtypeStruct((1, 1), jnp.float32),
  )(partials, W0p, b0, W1, b1, W2, b2)


@jax.jit
def kernel(x, pos, z, edge_index, W0, b0, W1, b1, W2, b2):
  feats = jnp.zeros((N_NODES, D_PAD), jnp.float32)
  feats = feats.at[:, :128].set(x)
  feats = feats.at[:, 128:131].set(pos)
  feats = feats.at[:, 131].set(z)

  src = edge_index[0].astype(jnp.int32)
  dst = edge_index[1].astype(jnp.int32)
  pad = E_PAD - N_EDGES
  src = jnp.concatenate([src, jnp.full((pad,), DUMMY_ROW, jnp.int32)])
  dst = jnp.concatenate([dst, jnp.zeros((pad,), jnp.int32)])
  src = src.reshape(NW, K_CHUNKS, CHUNK)
  dst = dst.reshape(NW, K_CHUNKS, CHUNK)
  zeros_blk = jnp.zeros((CHUNK, D_PAD), jnp.float32)

  out = _make_agg_kernel()(feats, src, dst, zeros_blk)
  partials = out.reshape(NUM_CORES, N_ACC, D_PAD)

  W0p = jnp.zeros((D_PAD, WIDTH), jnp.float32).at[:D_FEAT].set(W0)
  res = _mlp(partials, W0p, b0.reshape(1, WIDTH), W1, b1.reshape(1, WIDTH),
             W2, b2.reshape(1, 1))
  return res.reshape(1)


# speedup vs baseline: 5.2068x; 1.5442x over previous
import functools

import jax
import jax.numpy as jnp
from jax import lax
from jax.experimental import pallas as pl
from jax.experimental.pallas import tpu as pltpu
from jax.experimental.pallas import tpu_sc as plsc

N_NODES = 10000
N_EDGES = 320000
D_FEAT = 132
D_PAD = 144
D_HALF = 72
WIDTH = 128

NUM_CORES = 2
NUM_SUBCORES = 16

CHUNK = 128
K_CHUNKS = 160
NBUF = 4
E_PAD = NUM_SUBCORES * K_CHUNKS * CHUNK
ROWS_PER_TILE = 640
N_ACC = NUM_SUBCORES * ROWS_PER_TILE
DUMMY_ROW = N_NODES


def _make_agg_kernel():
  mesh = plsc.VectorSubcoreMesh(
      core_axis_name="c", subcore_axis_name="s",
      num_cores=NUM_CORES, num_subcores=NUM_SUBCORES)

  @functools.partial(
      pl.kernel,
      out_type=jax.ShapeDtypeStruct((NUM_CORES, N_ACC, D_HALF), jnp.float32),
      mesh=mesh,
      scratch_types=[
          pltpu.VMEM((K_CHUNKS, CHUNK), jnp.int32),
          pltpu.VMEM((K_CHUNKS, CHUNK), jnp.int32),
          [pltpu.VMEM((CHUNK, D_HALF), jnp.float32) for _ in range(NBUF)],
          pltpu.VMEM_SHARED((N_ACC, D_HALF), jnp.float32),
          [pltpu.SemaphoreType.DMA for _ in range(NBUF)],
          [pltpu.SemaphoreType.DMA for _ in range(NBUF)],
      ],
      compiler_params=pltpu.CompilerParams(use_tc_tiling_on_sc=False),
  )
  def agg(feats_hbm, src_hbm, dst_hbm, zeros_hbm, out_hbm,
          src_v, dst_v, bufs, accum, semg, sems):
    c = lax.axis_index("c")
    s = lax.axis_index("s")
    table = feats_hbm.at[c]

    pltpu.sync_copy(src_hbm.at[s], src_v)
    pltpu.sync_copy(dst_hbm.at[s], dst_v)

    pltpu.sync_copy(zeros_hbm, bufs[0])
    for k in range(ROWS_PER_TILE // CHUNK):
      pltpu.sync_copy(
          bufs[0], accum.at[pl.ds(s * ROWS_PER_TILE + k * CHUNK, CHUNK)])
    plsc.subcore_barrier()

    def gather(j, b):
      return pltpu.make_async_copy(table.at[dst_v.at[j]], bufs[b], semg[b])

    def scatter(j, b):
      return pltpu.make_async_copy(bufs[b], accum.at[src_v.at[j]], sems[b])

    for b in range(NBUF):
      pltpu.async_copy(table.at[dst_v.at[b]], bufs[b], semg[b])

    def body(g, carry):
      j0 = g * NBUF
      for b in range(NBUF):
        gather(j0 + b, b).wait()
        pltpu.async_copy(bufs[b], accum.at[src_v.at[j0 + b]], sems[b],
                         add=True)
      for b in range(NBUF):
        scatter(j0 + b, b).wait()
        pltpu.async_copy(table.at[dst_v.at[j0 + NBUF + b]], bufs[b], semg[b])
      return carry

    lax.fori_loop(0, K_CHUNKS // NBUF - 1, body, 0)
    j0 = K_CHUNKS - NBUF
    for b in range(NBUF):
      gather(j0 + b, b).wait()
      pltpu.async_copy(bufs[b], accum.at[src_v.at[j0 + b]], sems[b], add=True)
    for b in range(NBUF):
      scatter(j0 + b, b).wait()
    plsc.subcore_barrier()

    n_dump = ROWS_PER_TILE // CHUNK
    def rd(k, b):
      base = s * ROWS_PER_TILE + k * CHUNK
      return pltpu.make_async_copy(accum.at[pl.ds(base, CHUNK)], bufs[b],
                                   semg[b])
    def wr(k, b):
      base = s * ROWS_PER_TILE + k * CHUNK
      return pltpu.make_async_copy(bufs[b], out_hbm.at[c].at[pl.ds(base, CHUNK)],
                                   sems[b])
    for k in range(min(NBUF, n_dump)):
      rd(k, k).start()
    for k in range(n_dump):
      b = k % NBUF
      rd(k, b).wait()
      wr(k, b).start()
      if k + NBUF < n_dump:
        wr(k, b).wait()
        rd(k + NBUF, b).start()
    for k in range(max(0, n_dump - NBUF), n_dump):
      wr(k, k % NBUF).wait()

  return agg


def _mlp_body(p_ref, w0a_ref, w0b_ref, b0_ref, w1_ref, b1_ref, w2_ref, b2_ref,
              o_ref):
  sa = p_ref[0, :N_NODES]
  sb = p_ref[1, :N_NODES]
  nrm2 = (jnp.sum(sa * sa, axis=1, keepdims=True)
          + jnp.sum(sb * sb, axis=1, keepdims=True))
  inv = lax.rsqrt(nrm2)
  h = (jnp.dot(sa, w0a_ref[...], preferred_element_type=jnp.float32)
       + jnp.dot(sb, w0b_ref[...], preferred_element_type=jnp.float32))
  h = jnp.maximum(h * inv + b0_ref[...], 0.0)
  h = jnp.maximum(jnp.dot(h, w1_ref[...],
                          preferred_element_type=jnp.float32) + b1_ref[...], 0.0)
  r = jnp.dot(h, w2_ref[...], preferred_element_type=jnp.float32)
  o_ref[...] = (jnp.sum(r) / N_NODES + b2_ref[0, 0]).reshape(1, 1)


def _mlp(partials, W0a, W0b, b0, W1, b1, W2, b2):
  return pl.pallas_call(
      _mlp_body,
      out_shape=jax.ShapeDtypeStruct((1, 1), jnp.float32),
  )(partials, W0a, W0b, b0, W1, b1, W2, b2)


@jax.jit
def kernel(x, pos, z, edge_index, W0, b0, W1, b1, W2, b2):
  feats = jnp.zeros((NUM_CORES, N_NODES, D_HALF), jnp.float32)
  feats = feats.at[0].set(x[:, :D_HALF])
  feats = feats.at[1, :, :128 - D_HALF].set(x[:, D_HALF:])
  feats = feats.at[1, :, 56:59].set(pos)
  feats = feats.at[1, :, 59].set(z)

  src = edge_index[0].astype(jnp.int32)
  dst = edge_index[1].astype(jnp.int32)
  pad = E_PAD - N_EDGES
  src = jnp.concatenate([src, jnp.full((pad,), DUMMY_ROW, jnp.int32)])
  dst = jnp.concatenate([dst, jnp.zeros((pad,), jnp.int32)])
  src = src.reshape(K_CHUNKS, NUM_SUBCORES, CHUNK).swapaxes(0, 1)
  dst = dst.reshape(K_CHUNKS, NUM_SUBCORES, CHUNK).swapaxes(0, 1)
  zeros_blk = jnp.zeros((CHUNK, D_HALF), jnp.float32)

  partials = _make_agg_kernel()(feats, src, dst, zeros_blk)

  W0a = W0[:D_HALF]
  W0b = jnp.zeros((D_HALF, WIDTH), jnp.float32)
  W0b = W0b.at[:128 - D_HALF].set(W0[D_HALF:128])
  W0b = W0b.at[56:59].set(W0[128:131])
  W0b = W0b.at[59].set(W0[131])
  res = _mlp(partials, W0a, W0b, b0.reshape(1, WIDTH), W1, b1.reshape(1, WIDTH),
             W2, b2.reshape(1, 1))
  return res.reshape(1)


# speedup vs baseline: 5.2226x; 1.0030x over previous
import functools

import jax
import jax.numpy as jnp
from jax import lax
from jax.experimental import pallas as pl
from jax.experimental.pallas import tpu as pltpu
from jax.experimental.pallas import tpu_sc as plsc

N_NODES = 10000
N_EDGES = 320000
D_FEAT = 132
D_PAD = 144
D_HALF = 72
WIDTH = 128

NUM_CORES = 2
NUM_SUBCORES = 16

CHUNK = 128
K_CHUNKS = 160
NBUF = 4
E_PAD = NUM_SUBCORES * K_CHUNKS * CHUNK
ROWS_PER_TILE = 640
N_ACC = NUM_SUBCORES * ROWS_PER_TILE
DUMMY_ROW = N_NODES


def _make_agg_kernel():
  mesh = plsc.VectorSubcoreMesh(
      core_axis_name="c", subcore_axis_name="s",
      num_cores=NUM_CORES, num_subcores=NUM_SUBCORES)

  @functools.partial(
      pl.kernel,
      out_type=jax.ShapeDtypeStruct((NUM_CORES, N_ACC, D_HALF), jnp.float32),
      mesh=mesh,
      scratch_types=[
          pltpu.VMEM((K_CHUNKS, CHUNK), jnp.int32),
          pltpu.VMEM((K_CHUNKS, CHUNK), jnp.int32),
          [pltpu.VMEM((CHUNK, D_HALF), jnp.float32) for _ in range(NBUF)],
          pltpu.VMEM_SHARED((N_ACC, D_HALF), jnp.float32),
          [pltpu.SemaphoreType.DMA for _ in range(NBUF)],
          [pltpu.SemaphoreType.DMA for _ in range(NBUF)],
      ],
      compiler_params=pltpu.CompilerParams(use_tc_tiling_on_sc=False),
  )
  def agg(feats_hbm, src_hbm, dst_hbm, zeros_hbm, out_hbm,
          src_v, dst_v, bufs, accum, semg, sems):
    c = lax.axis_index("c")
    s = lax.axis_index("s")
    table = feats_hbm.at[c]

    pltpu.sync_copy(src_hbm.at[s], src_v)
    pltpu.sync_copy(dst_hbm.at[s], dst_v)

    pltpu.sync_copy(zeros_hbm, bufs[0])
    for k in range(ROWS_PER_TILE // CHUNK):
      pltpu.sync_copy(
          bufs[0], accum.at[pl.ds(s * ROWS_PER_TILE + k * CHUNK, CHUNK)])
    plsc.subcore_barrier()

    def gather(j, b):
      return pltpu.make_async_copy(table.at[dst_v.at[j]], bufs[b], semg[b])

    def scatter(j, b):
      return pltpu.make_async_copy(bufs[b], accum.at[src_v.at[j]], sems[b])

    for b in range(NBUF):
      pltpu.async_copy(table.at[dst_v.at[b]], bufs[b], semg[b])

    def body(g, carry):
      j0 = g * NBUF
      for b in range(NBUF):
        gather(j0 + b, b).wait()
        pltpu.async_copy(bufs[b], accum.at[src_v.at[j0 + b]], sems[b],
                         add=True)
      for b in range(NBUF):
        scatter(j0 + b, b).wait()
        pltpu.async_copy(table.at[dst_v.at[j0 + NBUF + b]], bufs[b], semg[b])
      return carry

    lax.fori_loop(0, K_CHUNKS // NBUF - 1, body, 0)
    j0 = K_CHUNKS - NBUF
    for b in range(NBUF):
      gather(j0 + b, b).wait()
      pltpu.async_copy(bufs[b], accum.at[src_v.at[j0 + b]], sems[b], add=True)
    for b in range(NBUF):
      scatter(j0 + b, b).wait()
    plsc.subcore_barrier()

    n_dump = ROWS_PER_TILE // CHUNK
    def rd(k, b):
      base = s * ROWS_PER_TILE + k * CHUNK
      return pltpu.make_async_copy(accum.at[pl.ds(base, CHUNK)], bufs[b],
                                   semg[b])
    def wr(k, b):
      base = s * ROWS_PER_TILE + k * CHUNK
      return pltpu.make_async_copy(bufs[b], out_hbm.at[c].at[pl.ds(base, CHUNK)],
                                   sems[b])
    for k in range(min(NBUF, n_dump)):
      rd(k, k).start()
    for k in range(n_dump):
      b = k % NBUF
      rd(k, b).wait()
      wr(k, b).start()
      if k + NBUF < n_dump:
        wr(k, b).wait()
        rd(k + NBUF, b).start()
    for k in range(max(0, n_dump - NBUF), n_dump):
      wr(k, k % NBUF).wait()

  return agg


def _mlp_body(p_ref, w0a_ref, w0b_ref, b0_ref, w1_ref, b1_ref, w2_ref, b2_ref,
              o_ref):
  sa = p_ref[0, :N_NODES]
  sb = p_ref[1, :N_NODES]
  nrm2 = (jnp.sum(sa * sa, axis=1, keepdims=True)
          + jnp.sum(sb * sb, axis=1, keepdims=True))
  inv = lax.rsqrt(nrm2)
  h = (jnp.dot(sa, w0a_ref[...], preferred_element_type=jnp.float32)
       + jnp.dot(sb, w0b_ref[...], preferred_element_type=jnp.float32))
  h = jnp.maximum(h * inv + b0_ref[...], 0.0)
  h = jnp.maximum(jnp.dot(h, w1_ref[...],
                          preferred_element_type=jnp.float32) + b1_ref[...], 0.0)
  r = jnp.dot(h, w2_ref[...], preferred_element_type=jnp.float32)
  o_ref[...] = (jnp.sum(r) / N_NODES + b2_ref[0, 0]).reshape(1, 1)


def _mlp(partials, W0a, W0b, b0, W1, b1, W2, b2):
  return pl.pallas_call(
      _mlp_body,
      out_shape=jax.ShapeDtypeStruct((1, 1), jnp.float32),
  )(partials, W0a, W0b, b0, W1, b1, W2, b2)


@jax.jit
def kernel(x, pos, z, edge_index, W0, b0, W1, b1, W2, b2):
  pad12 = jnp.zeros((N_NODES, D_HALF - 60), jnp.float32)
  feats = jnp.stack(
      [x[:, :D_HALF],
       jnp.concatenate([x[:, D_HALF:], pos, z[:, None], pad12], axis=1)])

  pad = E_PAD - N_EDGES
  pad_blk = jnp.tile(jnp.array([[DUMMY_ROW], [0]], jnp.int32), (1, pad))
  edges = jnp.concatenate([edge_index.astype(jnp.int32), pad_blk], axis=1)
  edges = edges.reshape(2, NUM_SUBCORES, K_CHUNKS, CHUNK)
  src = edges[0]
  dst = edges[1]
  zeros_blk = jnp.zeros((CHUNK, D_HALF), jnp.float32)

  partials = _make_agg_kernel()(feats, src, dst, zeros_blk)

  W0a = W0[:D_HALF]
  W0b = jnp.zeros((D_HALF, WIDTH), jnp.float32)
  W0b = W0b.at[:128 - D_HALF].set(W0[D_HALF:128])
  W0b = W0b.at[56:59].set(W0[128:131])
  W0b = W0b.at[59].set(W0[131])
  res = _mlp(partials, W0a, W0b, b0.reshape(1, WIDTH), W1, b1.reshape(1, WIDTH),
             W2, b2.reshape(1, 1))
  return res.reshape(1)


# speedup vs baseline: 5.2389x; 1.0031x over previous
import functools

import jax
import jax.numpy as jnp
from jax import lax
from jax.experimental import pallas as pl
from jax.experimental.pallas import tpu as pltpu
from jax.experimental.pallas import tpu_sc as plsc

N_NODES = 10000
N_EDGES = 320000
D_FEAT = 132
D_PAD = 144
D_HALF = 72
WIDTH = 128

NUM_CORES = 2
NUM_SUBCORES = 16

CHUNK = 128
K_CHUNKS = 160
NBUF = 4
E_PAD = NUM_SUBCORES * K_CHUNKS * CHUNK
ROWS_PER_TILE = 640
N_ACC = NUM_SUBCORES * ROWS_PER_TILE
DUMMY_ROW = N_NODES


def _make_agg_kernel():
  mesh = plsc.VectorSubcoreMesh(
      core_axis_name="c", subcore_axis_name="s",
      num_cores=NUM_CORES, num_subcores=NUM_SUBCORES)

  @functools.partial(
      pl.kernel,
      out_type=jax.ShapeDtypeStruct((NUM_CORES, N_ACC, D_HALF), jnp.float32),
      mesh=mesh,
      scratch_types=[
          pltpu.VMEM((K_CHUNKS, CHUNK), jnp.int32),
          pltpu.VMEM((K_CHUNKS, CHUNK), jnp.int32),
          [pltpu.VMEM((CHUNK, D_HALF), jnp.float32) for _ in range(NBUF)],
          pltpu.VMEM_SHARED((N_ACC, D_HALF), jnp.float32),
          [pltpu.SemaphoreType.DMA for _ in range(NBUF)],
          [pltpu.SemaphoreType.DMA for _ in range(NBUF)],
      ],
      compiler_params=pltpu.CompilerParams(use_tc_tiling_on_sc=False),
  )
  def agg(feats_hbm, src_hbm, dst_hbm, zeros_hbm, out_hbm,
          src_v, dst_v, bufs, accum, semg, sems):
    c = lax.axis_index("c")
    s = lax.axis_index("s")
    table = feats_hbm.at[c]

    pltpu.sync_copy(src_hbm.at[s], src_v)
    pltpu.sync_copy(dst_hbm.at[s], dst_v)

    pltpu.sync_copy(zeros_hbm, bufs[0])
    for k in range(ROWS_PER_TILE // CHUNK):
      pltpu.sync_copy(
          bufs[0], accum.at[pl.ds(s * ROWS_PER_TILE + k * CHUNK, CHUNK)])
    plsc.subcore_barrier()

    def gather(j, b):
      return pltpu.make_async_copy(table.at[dst_v.at[j]], bufs[b], semg[b])

    def scatter(j, b):
      return pltpu.make_async_copy(bufs[b], accum.at[src_v.at[j]], sems[b])

    for b in range(NBUF):
      pltpu.async_copy(table.at[dst_v.at[b]], bufs[b], semg[b])

    def body(g, carry):
      j0 = g * NBUF
      for b in range(NBUF):
        gather(j0 + b, b).wait()
        pltpu.async_copy(bufs[b], accum.at[src_v.at[j0 + b]], sems[b],
                         add=True)
      for b in range(NBUF):
        scatter(j0 + b, b).wait()
        pltpu.async_copy(table.at[dst_v.at[j0 + NBUF + b]], bufs[b], semg[b])
      return carry

    lax.fori_loop(0, K_CHUNKS // NBUF - 1, body, 0)
    j0 = K_CHUNKS - NBUF
    for b in range(NBUF):
      gather(j0 + b, b).wait()
      pltpu.async_copy(bufs[b], accum.at[src_v.at[j0 + b]], sems[b], add=True)
    for b in range(NBUF):
      scatter(j0 + b, b).wait()
    plsc.subcore_barrier()

    n_dump = ROWS_PER_TILE // CHUNK
    def rd(k, b):
      base = s * ROWS_PER_TILE + k * CHUNK
      return pltpu.make_async_copy(accum.at[pl.ds(base, CHUNK)], bufs[b],
                                   semg[b])
    def wr(k, b):
      base = s * ROWS_PER_TILE + k * CHUNK
      return pltpu.make_async_copy(bufs[b], out_hbm.at[c].at[pl.ds(base, CHUNK)],
                                   sems[b])
    for k in range(min(NBUF, n_dump)):
      rd(k, k).start()
    for k in range(n_dump):
      b = k % NBUF
      rd(k, b).wait()
      wr(k, b).start()
      if k + NBUF < n_dump:
        wr(k, b).wait()
        rd(k + NBUF, b).start()
    for k in range(max(0, n_dump - NBUF), n_dump):
      wr(k, k % NBUF).wait()

  return agg


def _mlp_body(p_ref, w0a_ref, w0b_ref, b0_ref, w1_ref, b1_ref, w2_ref, b2_ref,
              o_ref):
  sa = p_ref[0, :N_NODES]
  sb = p_ref[1, :N_NODES]
  nrm2 = (jnp.sum(sa * sa, axis=1, keepdims=True)
          + jnp.sum(sb * sb, axis=1, keepdims=True))
  inv = lax.rsqrt(nrm2)
  h = (jnp.dot(sa, w0a_ref[...], preferred_element_type=jnp.float32)
       + jnp.dot(sb, w0b_ref[...], preferred_element_type=jnp.float32))
  h = jnp.maximum(h * inv + b0_ref[...], 0.0)
  h = jnp.maximum(jnp.dot(h, w1_ref[...],
                          preferred_element_type=jnp.float32) + b1_ref[...], 0.0)
  r = jnp.dot(h, w2_ref[...], preferred_element_type=jnp.float32)
  o_ref[...] = (jnp.sum(r) / N_NODES + b2_ref[0, 0]).reshape(1, 1)


def _mlp(partials, W0a, W0b, b0, W1, b1, W2, b2):
  return pl.pallas_call(
      _mlp_body,
      out_shape=jax.ShapeDtypeStruct((1, 1), jnp.float32),
  )(partials, W0a, W0b, b0, W1, b1, W2, b2)


@jax.jit
def kernel(x, pos, z, edge_index, W0, b0, W1, b1, W2, b2):
  pad12 = jnp.zeros((N_NODES, D_HALF - 60), jnp.float32)
  feats = jnp.stack(
      [x[:, :D_HALF],
       jnp.concatenate([x[:, D_HALF:], pos, z[:, None], pad12], axis=1)])

  pad = E_PAD - N_EDGES
  pad_src = DUMMY_ROW + jnp.arange(pad, dtype=jnp.int32) % (N_ACC - DUMMY_ROW)
  pad_blk = jnp.stack([pad_src, jnp.zeros((pad,), jnp.int32)])
  edges = jnp.concatenate([edge_index.astype(jnp.int32), pad_blk], axis=1)
  edges = edges.reshape(2, NUM_SUBCORES, K_CHUNKS, CHUNK)
  src = edges[0]
  dst = edges[1]
  zeros_blk = jnp.zeros((CHUNK, D_HALF), jnp.float32)

  partials = _make_agg_kernel()(feats, src, dst, zeros_blk)

  W0a = W0[:D_HALF]
  W0b = jnp.zeros((D_HALF, WIDTH), jnp.float32)
  W0b = W0b.at[:128 - D_HALF].set(W0[D_HALF:128])
  W0b = W0b.at[56:59].set(W0[128:131])
  W0b = W0b.at[59].set(W0[131])
  res = _mlp(partials, W0a, W0b, b0.reshape(1, WIDTH), W1, b1.reshape(1, WIDTH),
             W2, b2.reshape(1, 1))
  return res.reshape(1)


# speedup vs baseline: 6.2481x; 1.1926x over previous
import functools

import jax
import jax.numpy as jnp
from jax import lax
from jax.experimental import pallas as pl
from jax.experimental.pallas import tpu as pltpu
from jax.experimental.pallas import tpu_sc as plsc

N_NODES = 10000
N_EDGES = 320000
D_FEAT = 132
D_PAD = 144
D_HALF = 72
WIDTH = 128

NUM_CORES = 2
NUM_SUBCORES = 16

CHUNK = 128
K_CHUNKS = 160
NBUF = 4
E_PAD = NUM_SUBCORES * K_CHUNKS * CHUNK
ROWS_PER_TILE = 640
N_ACC = NUM_SUBCORES * ROWS_PER_TILE
DUMMY_ROW = N_NODES


def _make_agg_kernel():
  mesh = plsc.VectorSubcoreMesh(
      core_axis_name="c", subcore_axis_name="s",
      num_cores=NUM_CORES, num_subcores=NUM_SUBCORES)

  @functools.partial(
      pl.kernel,
      out_type=jax.ShapeDtypeStruct((NUM_CORES, N_ACC, D_HALF), jnp.float32),
      mesh=mesh,
      scratch_types=[
          pltpu.VMEM((K_CHUNKS, CHUNK), jnp.int32),
          pltpu.VMEM((K_CHUNKS, CHUNK), jnp.int32),
          [pltpu.VMEM((CHUNK, D_HALF), jnp.float32) for _ in range(NBUF)],
          pltpu.VMEM_SHARED((N_ACC, D_HALF), jnp.float32),
          [pltpu.SemaphoreType.DMA for _ in range(NBUF)],
          [pltpu.SemaphoreType.DMA for _ in range(NBUF)],
      ],
      compiler_params=pltpu.CompilerParams(use_tc_tiling_on_sc=False),
  )
  def agg(feats_hbm, src_hbm, dst_hbm, zeros_hbm, out_hbm,
          src_v, dst_v, bufs, accum, semg, sems):
    c = lax.axis_index("c")
    s = lax.axis_index("s")
    table = feats_hbm.at[c]

    pltpu.sync_copy(src_hbm.at[s], src_v)
    pltpu.sync_copy(dst_hbm.at[s], dst_v)

    pltpu.sync_copy(zeros_hbm, bufs[0])
    for k in range(ROWS_PER_TILE // CHUNK):
      pltpu.sync_copy(
          bufs[0], accum.at[pl.ds(s * ROWS_PER_TILE + k * CHUNK, CHUNK)])
    plsc.subcore_barrier()

    def gather(j, b):
      return pltpu.make_async_copy(table.at[dst_v.at[j]], bufs[b], semg[b])

    def scatter(j, b):
      return pltpu.make_async_copy(bufs[b], accum.at[src_v.at[j]], sems[b])

    for b in range(NBUF):
      pltpu.async_copy(table.at[dst_v.at[b]], bufs[b], semg[b])

    def body(g, carry):
      j0 = g * NBUF
      for b in range(NBUF):
        gather(j0 + b, b).wait()
        pltpu.async_copy(bufs[b], accum.at[src_v.at[j0 + b]], sems[b],
                         add=True)
      for b in range(NBUF):
        scatter(j0 + b, b).wait()
        pltpu.async_copy(table.at[dst_v.at[j0 + NBUF + b]], bufs[b], semg[b])
      return carry

    lax.fori_loop(0, K_CHUNKS // NBUF - 1, body, 0)
    j0 = K_CHUNKS - NBUF
    for b in range(NBUF):
      gather(j0 + b, b).wait()
      pltpu.async_copy(bufs[b], accum.at[src_v.at[j0 + b]], sems[b], add=True)
    for b in range(NBUF):
      scatter(j0 + b, b).wait()
    plsc.subcore_barrier()

    n_dump = ROWS_PER_TILE // CHUNK
    def rd(k, b):
      base = s * ROWS_PER_TILE + k * CHUNK
      return pltpu.make_async_copy(accum.at[pl.ds(base, CHUNK)], bufs[b],
                                   semg[b])
    def wr(k, b):
      base = s * ROWS_PER_TILE + k * CHUNK
      return pltpu.make_async_copy(bufs[b], out_hbm.at[c].at[pl.ds(base, CHUNK)],
                                   sems[b])
    for k in range(min(NBUF, n_dump)):
      rd(k, k).start()
    for k in range(n_dump):
      b = k % NBUF
      rd(k, b).wait()
      wr(k, b).start()
      if k + NBUF < n_dump:
        wr(k, b).wait()
        rd(k + NBUF, b).start()
    for k in range(max(0, n_dump - NBUF), n_dump):
      wr(k, k % NBUF).wait()

  return agg


def _mlp_body(p_ref, w0a_ref, w0b_ref, b0_ref, w1_ref, b1_ref, w2_ref, b2_ref,
              o_ref):
  sa = p_ref[0, :N_NODES]
  sb = p_ref[1, :N_NODES]
  nrm2 = (jnp.sum(sa * sa, axis=1, keepdims=True)
          + jnp.sum(sb * sb, axis=1, keepdims=True))
  inv = lax.rsqrt(nrm2)
  h = (jnp.dot(sa, w0a_ref[...], preferred_element_type=jnp.float32)
       + jnp.dot(sb, w0b_ref[...], preferred_element_type=jnp.float32))
  h = jnp.maximum(h * inv + b0_ref[...], 0.0)
  h = jnp.maximum(jnp.dot(h, w1_ref[...],
                          preferred_element_type=jnp.float32) + b1_ref[...], 0.0)
  r = jnp.dot(h, w2_ref[...], preferred_element_type=jnp.float32)
  o_ref[...] = (jnp.sum(r) / N_NODES + b2_ref[0, 0]).reshape(1, 1)


def _mlp(partials, W0a, W0b, b0, W1, b1, W2, b2):
  return pl.pallas_call(
      _mlp_body,
      out_shape=jax.ShapeDtypeStruct((1, 1), jnp.float32),
  )(partials, W0a, W0b, b0, W1, b1, W2, b2)


@jax.jit
def kernel(x, pos, z, edge_index, W0, b0, W1, b1, W2, b2):
  pad12 = jnp.zeros((N_NODES, D_HALF - 60), jnp.float32)
  feats = jnp.stack(
      [x[:, :D_HALF],
       jnp.concatenate([x[:, D_HALF:], pos, z[:, None], pad12], axis=1)])

  pad = E_PAD - N_EDGES
  pad_src = DUMMY_ROW + jnp.arange(pad, dtype=jnp.int32) % (N_ACC - DUMMY_ROW)
  pad_blk = jnp.stack([pad_src, jnp.zeros((pad,), jnp.int32)])
  edges = jnp.concatenate([edge_index.astype(jnp.int32), pad_blk], axis=1)
  edges = edges.reshape(2, K_CHUNKS, NUM_SUBCORES, CHUNK).swapaxes(1, 2)
  src = edges[0]
  dst = edges[1]
  zeros_blk = jnp.zeros((CHUNK, D_HALF), jnp.float32)

  partials = _make_agg_kernel()(feats, src, dst, zeros_blk)

  W0a = W0[:D_HALF]
  W0b = jnp.zeros((D_HALF, WIDTH), jnp.float32)
  W0b = W0b.at[:128 - D_HALF].set(W0[D_HALF:128])
  W0b = W0b.at[56:59].set(W0[128:131])
  W0b = W0b.at[59].set(W0[131])
  res = _mlp(partials, W0a, W0b, b0.reshape(1, WIDTH), W1, b1.reshape(1, WIDTH),
             W2, b2.reshape(1, 1))
  return res.reshape(1)
